# baseline (speedup 1.0000x reference)
import jax
import jax.numpy as jnp
from jax.experimental import pallas as pl

N_NODE = 10000
N_EDGE = 320000
DIM = 128
N_REL = 16
N_LAYER = 6


def _ident_body(x_ref, o_ref):
    o_ref[...] = x_ref[...]


def _layer(x, boundary, edge_src, edge_dst, edge_rel, rel_e, W, b, g, beta):
    msg = rel_e[edge_rel] * x[edge_src]
    msg = jnp.concatenate([msg, boundary], axis=0)
    node_out = jnp.concatenate([edge_dst, jnp.arange(N_NODE, dtype=edge_dst.dtype)], axis=0)
    s = jax.ops.segment_sum(msg, node_out, num_segments=N_NODE)
    sq = jax.ops.segment_sum(msg * msg, node_out, num_segments=N_NODE)
    cnt = jax.ops.segment_sum(jnp.ones((msg.shape[0],), jnp.float32), node_out, num_segments=N_NODE)
    deg = cnt[:, None]
    mean = s / deg
    sq_mean = sq / deg
    mx = jax.ops.segment_max(msg, node_out, num_segments=N_NODE)
    mn = jax.ops.segment_min(msg, node_out, num_segments=N_NODE)
    std = jnp.sqrt(jnp.clip(sq_mean - mean ** 2, 1e-10, None))
    features = jnp.stack([mean, mx, mn, std], axis=-1).reshape(N_NODE, -1)
    scale = jnp.log(deg)
    scale = scale / scale.mean()
    scales = jnp.concatenate([jnp.ones_like(scale), scale, 1.0 / jnp.clip(scale, 1e-2, None)], axis=-1)
    update = (features[:, :, None] * scales[:, None, :]).reshape(N_NODE, -1)
    out = jnp.concatenate([x, update], axis=-1) @ W + b
    mu = out.mean(-1, keepdims=True)
    var = ((out - mu) ** 2).mean(-1, keepdims=True)
    out = (out - mu) / jnp.sqrt(var + 1e-5) * g + beta
    return jax.nn.relu(out)


def kernel(edge_index, edge_relation, h_index, rel_emb, W, b, ln_g, ln_b):
    edge_src = edge_index[:, 0]
    edge_dst = edge_index[:, 1]
    boundary = jnp.zeros((N_NODE, DIM), jnp.float32).at[h_index].set(1.0)
    x = boundary
    for i in range(N_LAYER):
        h = _layer(x, boundary, edge_src, edge_dst, edge_relation,
                   rel_emb[i], W[i], b[i], ln_g[i], ln_b[i])
        x = h + x
    x = pl.pallas_call(
        _ident_body,
        out_shape=jax.ShapeDtypeStruct((N_NODE, DIM), jnp.float32),
    )(x)
    return x


# speedup vs baseline: 7.1272x; 7.1272x over previous
import functools

import jax
import jax.numpy as jnp
from jax import lax
from jax.experimental import pallas as pl
from jax.experimental.pallas import tpu as pltpu
from jax.experimental.pallas import tpu_sc as plsc

N_NODE = 10000
N_EDGE = 320000
DIM = 128
N_REL = 16
N_LAYER = 6
N_QUERY = 64

NSEG = 128
NPS = 80
NPAD = NSEG * NPS
K = 120
CB = 128
EPAD = N_EDGE + CB
ROWB = 1024
GRID = NPAD // ROWB

_NEG = -3.0e38
_POS = 3.0e38


def _sc_edge_body(x_hbm, src_hbm, dst_hbm, rel_hbm, reltab_hbm, eb_hbm,
                  s_out, sq_out, mx_out, mn_out, cnt_out,
                  eb_v, rel_v, src_v, dst_v, relc_v, xrows_v,
                  s_acc, sq_acc, mx_acc, mn_acc, cnt_acc, sem):
    w = lax.axis_index("s") * 2 + lax.axis_index("c")
    pltpu.sync_copy(eb_hbm, eb_v)
    pltpu.sync_copy(reltab_hbm, rel_v)

    zv = jnp.zeros((16,), jnp.float32)
    negv = jnp.full((16,), _NEG, jnp.float32)
    posv = jnp.full((16,), _POS, jnp.float32)
    onehot0 = jnp.where(lax.iota(jnp.int32, 16) == 0, 1.0, 0.0)

    for t in range(NSEG // 32):
        seg = w * (NSEG // 32) + t
        n0 = seg * NPS
        ev = eb_v[pl.ds(seg, 16)]
        e0 = ev[0]
        e1 = ev[1]

        def init_row(r, _):
            for g in range(8):
                sl = pl.ds(g * 16, 16)
                s_acc[r, sl] = zv
                sq_acc[r, sl] = zv
                mx_acc[r, sl] = negv
                mn_acc[r, sl] = posv
            return 0
        lax.fori_loop(0, NPS, init_row, 0)
        for q in range((NPS + 16) // 16):
            cnt_acc[pl.ds(q * 16, 16)] = zv

        nch = (e1 - e0 + (K - 1)) // K

        def chunk_body(c, _):
            base = e0 + c * K
            base8 = (base // 8) * 8
            off = base - base8
            nedge = jnp.minimum(K, e1 - base)
            pltpu.sync_copy(src_hbm.at[pl.ds(base8, CB)], src_v)
            pltpu.sync_copy(dst_hbm.at[pl.ds(base8, CB)], dst_v.at[pl.ds(0, CB)])
            pltpu.sync_copy(rel_hbm.at[pl.ds(base8, CB)], relc_v.at[pl.ds(0, CB)])
            pltpu.async_copy(x_hbm.at[src_v], xrows_v, sem).wait()

            def edge_body(i, _):
                dl = dst_v[pl.ds(i, 16)][0] - n0
                r = relc_v[pl.ds(i, 16)][0]
                plsc.addupdate(cnt_acc.at[pl.ds(dl, 16)], onehot0)
                for g in range(8):
                    sl = pl.ds(g * 16, 16)
                    xv = xrows_v[i, sl]
                    rv = rel_v[r, sl]
                    msg = xv * rv
                    plsc.addupdate(s_acc.at[dl, sl], msg)
                    plsc.addupdate(sq_acc.at[dl, sl], msg * msg)
                    mx_acc[dl, sl] = jnp.maximum(mx_acc[dl, sl], msg)
                    mn_acc[dl, sl] = jnp.minimum(mn_acc[dl, sl], msg)
                return 0
            lax.fori_loop(off, off + nedge, edge_body, 0)
            return 0
        lax.fori_loop(0, nch, chunk_body, 0)

        pltpu.sync_copy(s_acc, s_out.at[pl.ds(n0, NPS)])
        pltpu.sync_copy(sq_acc, sq_out.at[pl.ds(n0, NPS)])
        pltpu.sync_copy(mx_acc, mx_out.at[pl.ds(n0, NPS)])
        pltpu.sync_copy(mn_acc, mn_out.at[pl.ds(n0, NPS)])
        pltpu.sync_copy(cnt_acc.at[pl.ds(0, NPS)], cnt_out.at[pl.ds(n0, NPS)])


_sc_edge = functools.partial(
    pl.kernel,
    mesh=plsc.VectorSubcoreMesh(core_axis_name="c", subcore_axis_name="s"),
    out_type=[
        jax.ShapeDtypeStruct((NPAD, DIM), jnp.float32),
        jax.ShapeDtypeStruct((NPAD, DIM), jnp.float32),
        jax.ShapeDtypeStruct((NPAD, DIM), jnp.float32),
        jax.ShapeDtypeStruct((NPAD, DIM), jnp.float32),
        jax.ShapeDtypeStruct((NPAD,), jnp.float32),
    ],
    scratch_types=[
        pltpu.VMEM((NSEG + 16,), jnp.int32),
        pltpu.VMEM((N_REL, DIM), jnp.float32),
        pltpu.VMEM((CB,), jnp.int32),
        pltpu.VMEM((CB + 16,), jnp.int32),
        pltpu.VMEM((CB + 16,), jnp.int32),
        pltpu.VMEM((CB, DIM), jnp.float32),
        pltpu.VMEM((NPS, DIM), jnp.float32),
        pltpu.VMEM((NPS, DIM), jnp.float32),
        pltpu.VMEM((NPS, DIM), jnp.float32),
        pltpu.VMEM((NPS, DIM), jnp.float32),
        pltpu.VMEM((NPS + 16,), jnp.float32),
        pltpu.SemaphoreType.DMA,
    ],
)(_sc_edge_body)


def _boundary_body(h_ref, o_ref):
    pid = pl.program_id(0)
    rows = lax.broadcasted_iota(jnp.int32, (ROWB, N_QUERY), 0) + pid * ROWB
    hit = jnp.where(rows == h_ref[...].astype(jnp.int32), 1.0, 0.0)
    m = jnp.max(hit, axis=1, keepdims=True)
    o_ref[...] = jnp.broadcast_to(m, (ROWB, DIM))


def _boundary(h2d):
    return pl.pallas_call(
        _boundary_body,
        grid=(GRID,),
        in_specs=[pl.BlockSpec((1, N_QUERY), lambda i: (0, 0))],
        out_specs=pl.BlockSpec((ROWB, DIM), lambda i: (i, 0)),
        out_shape=jax.ShapeDtypeStruct((NPAD, DIM), jnp.float32),
    )(h2d)


def _dense_body(x_ref, s_ref, sq_ref, mx_ref, mn_ref, cntb_ref, cnta_ref,
                bnd_ref, w_ref, b_ref, g_ref, beta_ref, o_ref):
    logd_all = jnp.log(cnta_ref[...] + 1.0)
    smean = jnp.sum(logd_all) * (1.0 / N_NODE)

    deg = cntb_ref[...] + 1.0
    bnd = bnd_ref[...]
    s = s_ref[...] + bnd
    sq = sq_ref[...] + bnd
    mean = s / deg
    sqm = sq / deg
    mx = jnp.maximum(mx_ref[...], bnd)
    mn = jnp.minimum(mn_ref[...], bnd)
    std = jnp.sqrt(jnp.clip(sqm - mean * mean, 1e-10, None))
    sc = jnp.log(deg) / smean
    isc = 1.0 / jnp.clip(sc, 1e-2, None)

    x = x_ref[...]
    cat = jnp.concatenate(
        [x, mean, mx, mn, std,
         mean * sc, mx * sc, mn * sc, std * sc,
         mean * isc, mx * isc, mn * isc, std * isc], axis=1)
    out = jnp.dot(cat, w_ref[...], preferred_element_type=jnp.float32)
    out = out + b_ref[...]
    mu = jnp.mean(out, axis=-1, keepdims=True)
    var = jnp.mean((out - mu) ** 2, axis=-1, keepdims=True)
    out = (out - mu) / jnp.sqrt(var + 1e-5) * g_ref[...] + beta_ref[...]
    o_ref[...] = jnp.maximum(out, 0.0) + x


def _dense(x, s, sq, mx, mn, cnt1, cnt2d, bnd, wp, b, g, beta):
    full = lambda i: (0, 0)
    row = lambda i: (i, 0)
    return pl.pallas_call(
        _dense_body,
        grid=(GRID,),
        in_specs=[
            pl.BlockSpec((ROWB, DIM), row),
            pl.BlockSpec((ROWB, DIM), row),
            pl.BlockSpec((ROWB, DIM), row),
            pl.BlockSpec((ROWB, DIM), row),
            pl.BlockSpec((ROWB, DIM), row),
            pl.BlockSpec((ROWB, 1), row),
            pl.BlockSpec((NPAD // DIM, DIM), full),
            pl.BlockSpec((ROWB, DIM), row),
            pl.BlockSpec((13 * DIM, DIM), full),
            pl.BlockSpec((1, DIM), full),
            pl.BlockSpec((1, DIM), full),
            pl.BlockSpec((1, DIM), full),
        ],
        out_specs=pl.BlockSpec((ROWB, DIM), row),
        out_shape=jax.ShapeDtypeStruct((NPAD, DIM), jnp.float32),
    )(x, s, sq, mx, mn, cnt1, cnt2d, bnd, wp, b, g, beta)


def kernel(edge_index, edge_relation, h_index, rel_emb, W, b, ln_g, ln_b):
    src = edge_index[:, 0].astype(jnp.int32)
    dst = edge_index[:, 1].astype(jnp.int32)
    rel = edge_relation.astype(jnp.int32)

    order = jnp.argsort(dst)
    srcp = jnp.concatenate([src[order], jnp.zeros((CB,), jnp.int32)])
    dstp = jnp.concatenate([dst[order], jnp.zeros((CB,), jnp.int32)])
    relp = jnp.concatenate([rel[order], jnp.zeros((CB,), jnp.int32)])
    marks = (jnp.arange(NSEG + 16, dtype=jnp.int32) * NPS)
    eb = jnp.searchsorted(dstp[:N_EDGE], marks, side="left").astype(jnp.int32)

    wu = W[:, DIM:, :].reshape(N_LAYER, DIM, 4, 3, DIM)
    wu = jnp.transpose(wu, (0, 3, 2, 1, 4)).reshape(N_LAYER, 12 * DIM, DIM)
    wp = jnp.concatenate([W[:, :DIM, :], wu], axis=1)

    h2d = h_index.astype(jnp.int32).reshape(1, N_QUERY)
    bnd = _boundary(h2d)
    b2 = b.reshape(N_LAYER, 1, DIM)
    g2 = ln_g.reshape(N_LAYER, 1, DIM)
    beta2 = ln_b.reshape(N_LAYER, 1, DIM)

    x = bnd
    for l in range(N_LAYER):
        s, sq, mx, mn, cnt = _sc_edge(x, srcp, dstp, relp,
                                      rel_emb[l], eb)
        cnt2d = cnt.reshape(NPAD // DIM, DIM)
        cnt1 = cnt.reshape(NPAD, 1)
        x = _dense(x, s, sq, mx, mn, cnt1, cnt2d, bnd,
                   wp[l], b2[l], g2[l], beta2[l])
    return x[:N_NODE]


# speedup vs baseline: 8.3050x; 1.1653x over previous
import functools

import jax
import jax.numpy as jnp
from jax import lax
from jax.experimental import pallas as pl
from jax.experimental.pallas import tpu as pltpu
from jax.experimental.pallas import tpu_sc as plsc

N_NODE = 10000
N_EDGE = 320000
DIM = 128
N_REL = 16
N_LAYER = 6
N_QUERY = 64

NSEG = 128
NPS = 80
NPAD = NSEG * NPS
K = 120
CB = 128
EPAD = N_EDGE + CB
ROWB = 1024
GRID = NPAD // ROWB

_NEG = -3.0e38
_POS = 3.0e38


def _sc_edge_body(x_hbm, src_hbm, e3_hbm, reltab_hbm, eb_hbm,
                  s_out, sq_out, mx_out, mn_out, cnt_out,
                  eb_v, rel_v, src_v0, src_v1, e3_v0, e3_v1,
                  xrows0, xrows1,
                  s_acc, sq_acc, mx_acc, mn_acc, cnt_acc,
                  sem_g0, sem_g1):
    w = lax.axis_index("s") * 2 + lax.axis_index("c")
    pltpu.sync_copy(eb_hbm, eb_v)
    pltpu.sync_copy(reltab_hbm, rel_v)

    src_bufs = (src_v0, src_v1)
    e3_bufs = (e3_v0, e3_v1)
    xr = (xrows0, xrows1)
    sem_g = (sem_g0, sem_g1)

    zv = jnp.zeros((16,), jnp.float32)
    negv = jnp.full((16,), _NEG, jnp.float32)
    posv = jnp.full((16,), _POS, jnp.float32)
    onehot0 = jnp.where(lax.iota(jnp.int32, 16) == 0, 1.0, 0.0)

    for t in range(NSEG // 32):
        seg = w * (NSEG // 32) + t
        n0 = seg * NPS
        ev = eb_v[pl.ds(seg, 16)]
        e0 = ev[0]
        e1 = ev[1]

        def init_row(r, _):
            for g in range(8):
                sl = pl.ds(g * 16, 16)
                s_acc[r, sl] = zv
                sq_acc[r, sl] = zv
                mx_acc[r, sl] = negv
                mn_acc[r, sl] = posv
            return 0
        lax.fori_loop(0, NPS, init_row, 0)
        for q in range((NPS + 16) // 16):
            cnt_acc[pl.ds(q * 16, 16)] = zv

        nch = (e1 - e0 + (K - 1)) // K

        def fetch(c, b):
            base = e0 + c * K
            base8 = (base // 8) * 8
            h1 = pltpu.async_copy(src_hbm.at[pl.ds(base8, CB)],
                                  src_bufs[b], sem_g[b])
            h2 = pltpu.async_copy(e3_hbm.at[pl.ds(base8 * 4, CB * 4)],
                                  e3_bufs[b].at[pl.ds(0, CB * 4)], sem_g[b])
            h1.wait()
            h2.wait()
            pltpu.async_copy(x_hbm.at[src_bufs[b]], xr[b], sem_g[b])

        def compute(c, b):
            base = e0 + c * K
            base8 = (base // 8) * 8
            off = base - base8
            nedge = jnp.minimum(K, e1 - base)
            xrows_v = xr[b]
            e3_v = e3_bufs[b]

            def edge_body(i, _):
                trip = e3_v[pl.ds(i * 4, 16)]
                dl = trip[1] - n0
                r = trip[2]
                plsc.addupdate(cnt_acc.at[pl.ds(dl, 16)], onehot0)
                for g in range(8):
                    sl = pl.ds(g * 16, 16)
                    xv = xrows_v[i, sl]
                    rv = rel_v[r, sl]
                    msg = xv * rv
                    plsc.addupdate(s_acc.at[dl, sl], msg)
                    plsc.addupdate(sq_acc.at[dl, sl], msg * msg)
                    mx_acc[dl, sl] = jnp.maximum(mx_acc[dl, sl], msg)
                    mn_acc[dl, sl] = jnp.minimum(mn_acc[dl, sl], msg)
                return 0
            lax.fori_loop(off, off + nedge, edge_body, 0)

        @pl.when(nch > 0)
        def _():
            fetch(0, 0)

        def pair_body(cc, _):
            for b in range(2):
                c = cc * 2 + b

                @pl.when(c < nch)
                def _(c=c, b=b):
                    @pl.when(c + 1 < nch)
                    def _():
                        fetch(c + 1, 1 - b)
                    pltpu.make_async_copy(x_hbm.at[src_bufs[b]],
                                          xr[b], sem_g[b]).wait()
                    compute(c, b)
            return 0
        lax.fori_loop(0, (nch + 1) // 2, pair_body, 0)

        pltpu.sync_copy(s_acc, s_out.at[pl.ds(n0, NPS)])
        pltpu.sync_copy(sq_acc, sq_out.at[pl.ds(n0, NPS)])
        pltpu.sync_copy(mx_acc, mx_out.at[pl.ds(n0, NPS)])
        pltpu.sync_copy(mn_acc, mn_out.at[pl.ds(n0, NPS)])
        pltpu.sync_copy(cnt_acc.at[pl.ds(0, NPS)], cnt_out.at[pl.ds(n0, NPS)])


_sc_edge = functools.partial(
    pl.kernel,
    mesh=plsc.VectorSubcoreMesh(core_axis_name="c", subcore_axis_name="s"),
    out_type=[
        jax.ShapeDtypeStruct((NPAD, DIM), jnp.float32),
        jax.ShapeDtypeStruct((NPAD, DIM), jnp.float32),
        jax.ShapeDtypeStruct((NPAD, DIM), jnp.float32),
        jax.ShapeDtypeStruct((NPAD, DIM), jnp.float32),
        jax.ShapeDtypeStruct((NPAD,), jnp.float32),
    ],
    scratch_types=[
        pltpu.VMEM((NSEG + 16,), jnp.int32),
        pltpu.VMEM((N_REL, DIM), jnp.float32),
        pltpu.VMEM((CB,), jnp.int32),
        pltpu.VMEM((CB,), jnp.int32),
        pltpu.VMEM((CB * 4 + 16,), jnp.int32),
        pltpu.VMEM((CB * 4 + 16,), jnp.int32),
        pltpu.VMEM((CB, DIM), jnp.float32),
        pltpu.VMEM((CB, DIM), jnp.float32),
        pltpu.VMEM((NPS, DIM), jnp.float32),
        pltpu.VMEM((NPS, DIM), jnp.float32),
        pltpu.VMEM((NPS, DIM), jnp.float32),
        pltpu.VMEM((NPS, DIM), jnp.float32),
        pltpu.VMEM((NPS + 16,), jnp.float32),
        pltpu.SemaphoreType.DMA,
        pltpu.SemaphoreType.DMA,
    ],
)(_sc_edge_body)


def _boundary_body(h_ref, o_ref):
    pid = pl.program_id(0)
    rows = lax.broadcasted_iota(jnp.int32, (ROWB, N_QUERY), 0) + pid * ROWB
    hit = jnp.where(rows == h_ref[...].astype(jnp.int32), 1.0, 0.0)
    m = jnp.max(hit, axis=1, keepdims=True)
    o_ref[...] = jnp.broadcast_to(m, (ROWB, DIM))


def _boundary(h2d):
    return pl.pallas_call(
        _boundary_body,
        grid=(GRID,),
        in_specs=[pl.BlockSpec((1, N_QUERY), lambda i: (0, 0))],
        out_specs=pl.BlockSpec((ROWB, DIM), lambda i: (i, 0)),
        out_shape=jax.ShapeDtypeStruct((NPAD, DIM), jnp.float32),
    )(h2d)


def _dense_body(x_ref, s_ref, sq_ref, mx_ref, mn_ref, cntb_ref, cnta_ref,
                bnd_ref, w_ref, b_ref, g_ref, beta_ref, o_ref):
    logd_all = jnp.log(cnta_ref[...] + 1.0)
    smean = jnp.sum(logd_all) * (1.0 / N_NODE)

    deg = cntb_ref[...] + 1.0
    bnd = bnd_ref[...]
    s = s_ref[...] + bnd
    sq = sq_ref[...] + bnd
    mean = s / deg
    sqm = sq / deg
    mx = jnp.maximum(mx_ref[...], bnd)
    mn = jnp.minimum(mn_ref[...], bnd)
    std = jnp.sqrt(jnp.clip(sqm - mean * mean, 1e-10, None))
    sc = jnp.log(deg) / smean
    isc = 1.0 / jnp.clip(sc, 1e-2, None)

    x = x_ref[...]
    cat = jnp.concatenate(
        [x, mean, mx, mn, std,
         mean * sc, mx * sc, mn * sc, std * sc,
         mean * isc, mx * isc, mn * isc, std * isc], axis=1)
    out = jnp.dot(cat, w_ref[...], preferred_element_type=jnp.float32)
    out = out + b_ref[...]
    mu = jnp.mean(out, axis=-1, keepdims=True)
    var = jnp.mean((out - mu) ** 2, axis=-1, keepdims=True)
    out = (out - mu) / jnp.sqrt(var + 1e-5) * g_ref[...] + beta_ref[...]
    o_ref[...] = jnp.maximum(out, 0.0) + x


def _dense(x, s, sq, mx, mn, cnt1, cnt2d, bnd, wp, b, g, beta):
    full = lambda i: (0, 0)
    row = lambda i: (i, 0)
    return pl.pallas_call(
        _dense_body,
        grid=(GRID,),
        in_specs=[
            pl.BlockSpec((ROWB, DIM), row),
            pl.BlockSpec((ROWB, DIM), row),
            pl.BlockSpec((ROWB, DIM), row),
            pl.BlockSpec((ROWB, DIM), row),
            pl.BlockSpec((ROWB, DIM), row),
            pl.BlockSpec((ROWB, 1), row),
            pl.BlockSpec((NPAD // DIM, DIM), full),
            pl.BlockSpec((ROWB, DIM), row),
            pl.BlockSpec((13 * DIM, DIM), full),
            pl.BlockSpec((1, DIM), full),
            pl.BlockSpec((1, DIM), full),
            pl.BlockSpec((1, DIM), full),
        ],
        out_specs=pl.BlockSpec((ROWB, DIM), row),
        out_shape=jax.ShapeDtypeStruct((NPAD, DIM), jnp.float32),
    )(x, s, sq, mx, mn, cnt1, cnt2d, bnd, wp, b, g, beta)


def kernel(edge_index, edge_relation, h_index, rel_emb, W, b, ln_g, ln_b):
    src = edge_index[:, 0].astype(jnp.int32)
    dst = edge_index[:, 1].astype(jnp.int32)
    rel = edge_relation.astype(jnp.int32)

    order = jnp.argsort(dst)
    srcp = jnp.concatenate([src[order], jnp.zeros((CB,), jnp.int32)])
    dstp = jnp.concatenate([dst[order], jnp.zeros((CB,), jnp.int32)])
    relp = jnp.concatenate([rel[order], jnp.zeros((CB,), jnp.int32)])
    e3 = jnp.stack([srcp, dstp, relp, jnp.zeros_like(srcp)], axis=1).reshape(-1)
    marks = (jnp.arange(NSEG + 16, dtype=jnp.int32) * NPS)
    eb = jnp.searchsorted(dstp[:N_EDGE], marks, side="left").astype(jnp.int32)

    wu = W[:, DIM:, :].reshape(N_LAYER, DIM, 4, 3, DIM)
    wu = jnp.transpose(wu, (0, 3, 2, 1, 4)).reshape(N_LAYER, 12 * DIM, DIM)
    wp = jnp.concatenate([W[:, :DIM, :], wu], axis=1)

    h2d = h_index.astype(jnp.int32).reshape(1, N_QUERY)
    bnd = _boundary(h2d)
    b2 = b.reshape(N_LAYER, 1, DIM)
    g2 = ln_g.reshape(N_LAYER, 1, DIM)
    beta2 = ln_b.reshape(N_LAYER, 1, DIM)

    x = bnd
    for l in range(N_LAYER):
        s, sq, mx, mn, cnt = _sc_edge(x, srcp, e3, rel_emb[l], eb)
        cnt2d = cnt.reshape(NPAD // DIM, DIM)
        cnt1 = cnt.reshape(NPAD, 1)
        x = _dense(x, s, sq, mx, mn, cnt1, cnt2d, bnd,
                   wp[l], b2[l], g2[l], beta2[l])
    return x[:N_NODE]


# speedup vs baseline: 9.1652x; 1.1036x over previous
import functools

import jax
import jax.numpy as jnp
from jax import lax
from jax.experimental import pallas as pl
from jax.experimental.pallas import tpu as pltpu
from jax.experimental.pallas import tpu_sc as plsc

N_NODE = 10000
N_EDGE = 320000
DIM = 128
N_REL = 16
N_LAYER = 6
N_QUERY = 64

NSEG = 128
NPS = 80
NPAD = NSEG * NPS
K = 120
CB = 128
EPAD = N_EDGE + CB
ROWB = 1024
GRID = NPAD // ROWB

_NEG = -3.0e38
_POS = 3.0e38


def _sc_edge_body(x_hbm, src_hbm, e3_hbm, reltab_hbm, eb_hbm,
                  s_out, sq_out, mx_out, mn_out, cnt_out,
                  eb_v, rel_v, src_v0, src_v1, e3_v0, e3_v1,
                  xrows0, xrows1,
                  s_acc, sq_acc, mx_acc, mn_acc, cnt_acc,
                  sem_g0, sem_g1):
    w = lax.axis_index("s") * 2 + lax.axis_index("c")
    pltpu.sync_copy(eb_hbm, eb_v)
    pltpu.sync_copy(reltab_hbm, rel_v)

    src_bufs = (src_v0, src_v1)
    e3_bufs = (e3_v0, e3_v1)
    xr = (xrows0, xrows1)
    sem_g = (sem_g0, sem_g1)

    zv = jnp.zeros((16,), jnp.float32)
    negv = jnp.full((16,), _NEG, jnp.float32)
    posv = jnp.full((16,), _POS, jnp.float32)
    onehot0 = jnp.where(lax.iota(jnp.int32, 16) == 0, 1.0, 0.0)

    for t in range(NSEG // 32):
        seg = w * (NSEG // 32) + t
        n0 = seg * NPS
        ev = eb_v[pl.ds(seg, 16)]
        e0 = ev[0]
        e1 = ev[1]

        def init_row(r, _):
            for g in range(8):
                sl = pl.ds(g * 16, 16)
                s_acc[r, sl] = zv
                sq_acc[r, sl] = zv
                mx_acc[r, sl] = negv
                mn_acc[r, sl] = posv
            return 0
        lax.fori_loop(0, NPS, init_row, 0)
        for q in range((NPS + 16) // 16):
            cnt_acc[pl.ds(q * 16, 16)] = zv

        nch = (e1 - e0 + (K - 1)) // K

        def fetch(c, b):
            base = e0 + c * K
            base8 = (base // 8) * 8
            h1 = pltpu.async_copy(src_hbm.at[pl.ds(base8, CB)],
                                  src_bufs[b], sem_g[b])
            h2 = pltpu.async_copy(e3_hbm.at[pl.ds(base8 * 4, CB * 4)],
                                  e3_bufs[b].at[pl.ds(0, CB * 4)], sem_g[b])
            h1.wait()
            h2.wait()
            pltpu.async_copy(x_hbm.at[src_bufs[b]], xr[b], sem_g[b])

        def compute(c, b):
            base = e0 + c * K
            base8 = (base // 8) * 8
            off = base - base8
            nedge = jnp.minimum(K, e1 - base)
            xrows_v = xr[b]
            e3_v = e3_bufs[b]
            lim = off + nedge

            def quad_body(q, _):
                i0 = off + q * 4
                meta = e3_v[pl.ds(i0 * 4, 16)]
                for j in range(4):
                    i = i0 + j
                    valid = i < lim
                    dl = jnp.where(valid, meta[j * 4 + 1] - n0, NPS)
                    r = jnp.where(valid, meta[j * 4 + 2], 0)
                    plsc.addupdate(cnt_acc.at[pl.ds(dl, 16)], onehot0)
                    for g in range(8):
                        sl = pl.ds(g * 16, 16)
                        xv = xrows_v[i, sl]
                        rv = rel_v[r, sl]
                        msg = xv * rv
                        plsc.addupdate(s_acc.at[dl, sl], msg)
                        plsc.addupdate(sq_acc.at[dl, sl], msg * msg)
                        mx_acc[dl, sl] = jnp.maximum(mx_acc[dl, sl], msg)
                        mn_acc[dl, sl] = jnp.minimum(mn_acc[dl, sl], msg)
                return 0
            lax.fori_loop(0, (nedge + 3) // 4, quad_body, 0)

        @pl.when(nch > 0)
        def _():
            fetch(0, 0)

        def pair_body(cc, _):
            for b in range(2):
                c = cc * 2 + b

                @pl.when(c < nch)
                def _(c=c, b=b):
                    @pl.when(c + 1 < nch)
                    def _():
                        fetch(c + 1, 1 - b)
                    pltpu.make_async_copy(x_hbm.at[src_bufs[b]],
                                          xr[b], sem_g[b]).wait()
                    compute(c, b)
            return 0
        lax.fori_loop(0, (nch + 1) // 2, pair_body, 0)

        pltpu.sync_copy(s_acc.at[pl.ds(0, NPS)], s_out.at[pl.ds(n0, NPS)])
        pltpu.sync_copy(sq_acc.at[pl.ds(0, NPS)], sq_out.at[pl.ds(n0, NPS)])
        pltpu.sync_copy(mx_acc.at[pl.ds(0, NPS)], mx_out.at[pl.ds(n0, NPS)])
        pltpu.sync_copy(mn_acc.at[pl.ds(0, NPS)], mn_out.at[pl.ds(n0, NPS)])
        pltpu.sync_copy(cnt_acc.at[pl.ds(0, NPS)], cnt_out.at[pl.ds(n0, NPS)])


_sc_edge = functools.partial(
    pl.kernel,
    mesh=plsc.VectorSubcoreMesh(core_axis_name="c", subcore_axis_name="s"),
    out_type=[
        jax.ShapeDtypeStruct((NPAD, DIM), jnp.float32),
        jax.ShapeDtypeStruct((NPAD, DIM), jnp.float32),
        jax.ShapeDtypeStruct((NPAD, DIM), jnp.float32),
        jax.ShapeDtypeStruct((NPAD, DIM), jnp.float32),
        jax.ShapeDtypeStruct((NPAD,), jnp.float32),
    ],
    scratch_types=[
        pltpu.VMEM((NSEG + 16,), jnp.int32),
        pltpu.VMEM((N_REL, DIM), jnp.float32),
        pltpu.VMEM((CB,), jnp.int32),
        pltpu.VMEM((CB,), jnp.int32),
        pltpu.VMEM((CB * 4 + 16,), jnp.int32),
        pltpu.VMEM((CB * 4 + 16,), jnp.int32),
        pltpu.VMEM((CB, DIM), jnp.float32),
        pltpu.VMEM((CB, DIM), jnp.float32),
        pltpu.VMEM((NPS + 8, DIM), jnp.float32),
        pltpu.VMEM((NPS + 8, DIM), jnp.float32),
        pltpu.VMEM((NPS + 8, DIM), jnp.float32),
        pltpu.VMEM((NPS + 8, DIM), jnp.float32),
        pltpu.VMEM((NPS + 16,), jnp.float32),
        pltpu.SemaphoreType.DMA,
        pltpu.SemaphoreType.DMA,
    ],
)(_sc_edge_body)


def _boundary_body(h_ref, o_ref):
    pid = pl.program_id(0)
    rows = lax.broadcasted_iota(jnp.int32, (ROWB, N_QUERY), 0) + pid * ROWB
    hit = jnp.where(rows == h_ref[...].astype(jnp.int32), 1.0, 0.0)
    m = jnp.max(hit, axis=1, keepdims=True)
    o_ref[...] = jnp.broadcast_to(m, (ROWB, DIM))


def _boundary(h2d):
    return pl.pallas_call(
        _boundary_body,
        grid=(GRID,),
        in_specs=[pl.BlockSpec((1, N_QUERY), lambda i: (0, 0))],
        out_specs=pl.BlockSpec((ROWB, DIM), lambda i: (i, 0)),
        out_shape=jax.ShapeDtypeStruct((NPAD, DIM), jnp.float32),
    )(h2d)


def _dense_body(x_ref, s_ref, sq_ref, mx_ref, mn_ref, cntb_ref, cnta_ref,
                bnd_ref, w_ref, b_ref, g_ref, beta_ref, o_ref):
    logd_all = jnp.log(cnta_ref[...] + 1.0)
    smean = jnp.sum(logd_all) * (1.0 / N_NODE)

    deg = cntb_ref[...] + 1.0
    bnd = bnd_ref[...]
    s = s_ref[...] + bnd
    sq = sq_ref[...] + bnd
    mean = s / deg
    sqm = sq / deg
    mx = jnp.maximum(mx_ref[...], bnd)
    mn = jnp.minimum(mn_ref[...], bnd)
    std = jnp.sqrt(jnp.clip(sqm - mean * mean, 1e-10, None))
    sc = jnp.log(deg) / smean
    isc = 1.0 / jnp.clip(sc, 1e-2, None)

    x = x_ref[...]
    cat = jnp.concatenate(
        [x, mean, mx, mn, std,
         mean * sc, mx * sc, mn * sc, std * sc,
         mean * isc, mx * isc, mn * isc, std * isc], axis=1)
    out = jnp.dot(cat, w_ref[...], preferred_element_type=jnp.float32)
    out = out + b_ref[...]
    mu = jnp.mean(out, axis=-1, keepdims=True)
    var = jnp.mean((out - mu) ** 2, axis=-1, keepdims=True)
    out = (out - mu) / jnp.sqrt(var + 1e-5) * g_ref[...] + beta_ref[...]
    o_ref[...] = jnp.maximum(out, 0.0) + x


def _dense(x, s, sq, mx, mn, cnt1, cnt2d, bnd, wp, b, g, beta):
    full = lambda i: (0, 0)
    row = lambda i: (i, 0)
    return pl.pallas_call(
        _dense_body,
        grid=(GRID,),
        in_specs=[
            pl.BlockSpec((ROWB, DIM), row),
            pl.BlockSpec((ROWB, DIM), row),
            pl.BlockSpec((ROWB, DIM), row),
            pl.BlockSpec((ROWB, DIM), row),
            pl.BlockSpec((ROWB, DIM), row),
            pl.BlockSpec((ROWB, 1), row),
            pl.BlockSpec((NPAD // DIM, DIM), full),
            pl.BlockSpec((ROWB, DIM), row),
            pl.BlockSpec((13 * DIM, DIM), full),
            pl.BlockSpec((1, DIM), full),
            pl.BlockSpec((1, DIM), full),
            pl.BlockSpec((1, DIM), full),
        ],
        out_specs=pl.BlockSpec((ROWB, DIM), row),
        out_shape=jax.ShapeDtypeStruct((NPAD, DIM), jnp.float32),
    )(x, s, sq, mx, mn, cnt1, cnt2d, bnd, wp, b, g, beta)


def kernel(edge_index, edge_relation, h_index, rel_emb, W, b, ln_g, ln_b):
    src = edge_index[:, 0].astype(jnp.int32)
    dst = edge_index[:, 1].astype(jnp.int32)
    rel = edge_relation.astype(jnp.int32)

    order = jnp.argsort(dst)
    srcp = jnp.concatenate([src[order], jnp.zeros((CB,), jnp.int32)])
    dstp = jnp.concatenate([dst[order], jnp.zeros((CB,), jnp.int32)])
    relp = jnp.concatenate([rel[order], jnp.zeros((CB,), jnp.int32)])
    e3 = jnp.stack([srcp, dstp, relp, jnp.zeros_like(srcp)], axis=1).reshape(-1)
    marks = (jnp.arange(NSEG + 16, dtype=jnp.int32) * NPS)
    eb = jnp.searchsorted(dstp[:N_EDGE], marks, side="left").astype(jnp.int32)

    wu = W[:, DIM:, :].reshape(N_LAYER, DIM, 4, 3, DIM)
    wu = jnp.transpose(wu, (0, 3, 2, 1, 4)).reshape(N_LAYER, 12 * DIM, DIM)
    wp = jnp.concatenate([W[:, :DIM, :], wu], axis=1)

    h2d = h_index.astype(jnp.int32).reshape(1, N_QUERY)
    bnd = _boundary(h2d)
    b2 = b.reshape(N_LAYER, 1, DIM)
    g2 = ln_g.reshape(N_LAYER, 1, DIM)
    beta2 = ln_b.reshape(N_LAYER, 1, DIM)

    x = bnd
    for l in range(N_LAYER):
        s, sq, mx, mn, cnt = _sc_edge(x, srcp, e3, rel_emb[l], eb)
        cnt2d = cnt.reshape(NPAD // DIM, DIM)
        cnt1 = cnt.reshape(NPAD, 1)
        x = _dense(x, s, sq, mx, mn, cnt1, cnt2d, bnd,
                   wp[l], b2[l], g2[l], beta2[l])
    return x[:N_NODE]
